# baseline (speedup 1.0000x reference)
import functools

import jax
import jax.numpy as jnp
from jax import lax
from jax.experimental import pallas as pl
from jax.experimental.pallas import tpu as pltpu
from jax.experimental.pallas import tpu_sc as plsc

N = 10000
NP = 10112
E = 320000
FEAT = 128
G = 128
NC = 2
NS = 16
NW = NC * NS
CHUNK = 128
NCH = 80
EP = NW * NCH * CHUNK
RPS = NP // NS


def _sc_agg_body(h_hbm, idx_hbm, out_hbm,
                 ib_a, ib_b, rows_a, rows_b, zv, acc_sh,
                 sem_a, sem_b, sem_ia, sem_ib):
    c = lax.axis_index("c")
    s = lax.axis_index("s")
    wid = s * NC + c

    zero16 = jnp.zeros((16,), jnp.float32)
    for r in range(16):
        for q in range(8):
            zv[r, pl.ds(q * 16, 16)] = zero16
    row0 = s * RPS

    @pl.loop(0, 39)
    def _zero(i):
        pltpu.sync_copy(zv, acc_sh.at[pl.ds(row0 + i * 16, 16)])

    pltpu.sync_copy(zv.at[pl.ds(0, 8)], acc_sh.at[pl.ds(row0 + 624, 8)])
    plsc.subcore_barrier()

    def _fire_idx(p, ib, sem):
        return pltpu.async_copy(idx_hbm.at[wid, p], ib, sem)

    def _wait_idx(p, ib, sem):
        pltpu.make_async_copy(idx_hbm.at[wid, p], ib, sem).wait()

    def _fire_gather(ib, r, rows, sem):
        return pltpu.async_copy(h_hbm.at[ib.at[r]], rows, sem)

    def _drain(ib, r, rows, sem):
        pltpu.make_async_copy(h_hbm.at[ib.at[r - 1]], rows, sem).wait()
        pltpu.sync_copy(rows, acc_sh.at[ib.at[r]], add=True)

    npair = NCH // 2
    _fire_idx(0, ib_a, sem_ia)
    _wait_idx(0, ib_a, sem_ia)
    _fire_gather(ib_a, 0, rows_a, sem_a)
    _fire_idx(1, ib_b, sem_ib)

    def _pair(p, A, B, sa, sb, sia, sib):
        _fire_gather(A, 2, rows_b, sb)
        _drain(A, 1, rows_a, sa)
        _wait_idx(p + 1, B, sib)
        _fire_gather(B, 0, rows_a, sa)
        _drain(A, 3, rows_b, sb)
        _fire_idx(p + 2, A, sia)

    @pl.loop(0, npair // 2 - 1)
    def _edge(k):
        p = 2 * k
        _pair(p, ib_a, ib_b, sem_a, sem_b, sem_ia, sem_ib)
        _pair(p + 1, ib_b, ib_a, sem_a, sem_b, sem_ib, sem_ia)

    _fire_gather(ib_a, 2, rows_b, sem_b)
    _drain(ib_a, 1, rows_a, sem_a)
    _wait_idx(npair - 1, ib_b, sem_ib)
    _fire_gather(ib_b, 0, rows_a, sem_a)
    _drain(ib_a, 3, rows_b, sem_b)
    _fire_gather(ib_b, 2, rows_b, sem_b)
    _drain(ib_b, 1, rows_a, sem_a)
    _drain(ib_b, 3, rows_b, sem_b)

    plsc.subcore_barrier()
    pltpu.sync_copy(acc_sh.at[pl.ds(row0, RPS)],
                    out_hbm.at[c, pl.ds(row0, RPS)])


_sc_agg = functools.partial(
    pl.kernel,
    out_type=jax.ShapeDtypeStruct((NC, NP, FEAT), jnp.float32),
    mesh=plsc.VectorSubcoreMesh(core_axis_name="c", subcore_axis_name="s"),
    scratch_types=[
        pltpu.VMEM((4, CHUNK), jnp.int32),
        pltpu.VMEM((4, CHUNK), jnp.int32),
        pltpu.VMEM((CHUNK, FEAT), jnp.float32),
        pltpu.VMEM((CHUNK, FEAT), jnp.float32),
        pltpu.VMEM((16, FEAT), jnp.float32),
        pltpu.VMEM_SHARED((NP, FEAT), jnp.float32),
        pltpu.SemaphoreType.DMA,
        pltpu.SemaphoreType.DMA,
        pltpu.SemaphoreType.DMA,
        pltpu.SemaphoreType.DMA,
    ],
)(_sc_agg_body)


def _tc_input_body(x_ref, w_ref, b_ref, o_ref):
    h = jnp.dot(x_ref[...], w_ref[...], preferred_element_type=jnp.float32)
    o_ref[0:N, :] = jnp.maximum(h + b_ref[...], 0.0)


def _tc_input(x, w0, b0):
    return pl.pallas_call(
        _tc_input_body,
        out_shape=jax.ShapeDtypeStruct((NP, FEAT), jnp.float32),
    )(x, w0, b0)


def _bn_relu(z, g, b):
    m = jnp.sum(z, axis=0, keepdims=True) * (1.0 / N)
    d = z - m
    v = jnp.sum(d * d, axis=0, keepdims=True) * (1.0 / N)
    return jnp.maximum(g * d * jax.lax.rsqrt(v + 1e-5) + b, 0.0)


def _tc_dense_body(h_ref, p_ref, eps_ref, w1_ref, b1_ref, g1_ref, be1_ref,
                   w2_ref, b2_ref, g2_ref, be2_ref, o_ref):
    h = h_ref[0:N, :]
    agg = p_ref[0, 0:N, :] + p_ref[1, 0:N, :]
    z = (1.0 + eps_ref[0, 0]) * h + agg
    z = jnp.dot(z, w1_ref[...], preferred_element_type=jnp.float32) + b1_ref[...]
    z = _bn_relu(z, g1_ref[...], be1_ref[...])
    z = jnp.dot(z, w2_ref[...], preferred_element_type=jnp.float32) + b2_ref[...]
    z = _bn_relu(z, g2_ref[...], be2_ref[...])
    o_ref[0:N, :] = z + h


def _tc_dense(h, parts, eps_l, w1, b1, g1, be1, w2, b2, g2, be2):
    return pl.pallas_call(
        _tc_dense_body,
        out_shape=jax.ShapeDtypeStruct((NP, FEAT), jnp.float32),
    )(h, parts, eps_l, w1, b1, g1, be1, w2, b2, g2, be2)


def _tc_pool_body(h_ref, batch_ref, wout_ref, bout_ref, o_ref):
    h = h_ref[0:N, :]
    ids = batch_ref[...]
    onehot = (ids == lax.broadcasted_iota(jnp.int32, (N, G), 1))
    onehot = onehot.astype(jnp.float32)
    cdims = (((0,), (0,)), ((), ()))
    sums = lax.dot_general(onehot, h, cdims,
                           preferred_element_type=jnp.float32)
    cnt = lax.dot_general(onehot, jnp.ones((N, 1), jnp.float32), cdims,
                          preferred_element_type=jnp.float32)
    pooled = sums / jnp.maximum(cnt, 1.0)
    o_ref[...] = jnp.dot(pooled, wout_ref[...],
                         preferred_element_type=jnp.float32) + bout_ref[...]


def _tc_pool(h, batch2d, wout, bout):
    return pl.pallas_call(
        _tc_pool_body,
        out_shape=jax.ShapeDtypeStruct((G, FEAT), jnp.float32),
    )(h, batch2d, wout, bout)


def kernel(x, edge_index, edge_attr, batch, W0, b0, eps, W1s, b1s, g1s, be1s,
           W2s, b2s, gs, bes, Wout, bout):
    pad = EP - E
    src = jnp.concatenate([edge_index[0], jnp.full((pad,), N, jnp.int32)])
    dst = jnp.concatenate([edge_index[1], jnp.full((pad,), N, jnp.int32)])
    src = src.reshape(NW, NCH, CHUNK)
    dst = dst.reshape(NW, NCH, CHUNK)
    idx = jnp.stack([src[:, 0::2], dst[:, 0::2], src[:, 1::2], dst[:, 1::2]],
                    axis=2)

    h = _tc_input(x, W0, b0.reshape(1, FEAT))
    for l in range(2):
        parts = _sc_agg(h, idx)
        h = _tc_dense(h, parts, eps[l].reshape(1, 1),
                      W1s[l], b1s[l].reshape(1, FEAT),
                      g1s[l].reshape(1, FEAT), be1s[l].reshape(1, FEAT),
                      W2s[l], b2s[l].reshape(1, FEAT),
                      gs[l].reshape(1, FEAT), bes[l].reshape(1, FEAT))
    return _tc_pool(h, batch.reshape(N, 1).astype(jnp.int32),
                    Wout, bout.reshape(1, FEAT))


# speedup vs baseline: 3.6986x; 3.6986x over previous
import functools

import jax
import jax.numpy as jnp
from jax import lax
from jax.experimental import pallas as pl
from jax.experimental.pallas import tpu as pltpu
from jax.experimental.pallas import tpu_sc as plsc

N = 10000
NP = 10112
E = 320000
FEAT = 128
G = 128
NC = 2
NS = 16
NW = NC * NS
CHUNK = 128
NCH = 80
EP = NW * NCH * CHUNK
RPS = NP // NS


def _sc_agg_body(h_hbm, idx_hbm, out_hbm,
                 ib_a, ib_b, rows_a, rows_b, zv, acc_sh,
                 sem_a, sem_b, sem_ia, sem_ib):
    c = lax.axis_index("c")
    s = lax.axis_index("s")
    wid = s * NC + c

    zero16 = jnp.zeros((16,), jnp.float32)
    for r in range(16):
        for q in range(8):
            zv[r, pl.ds(q * 16, 16)] = zero16
    row0 = s * RPS

    @pl.loop(0, 39)
    def _zero(i):
        pltpu.sync_copy(zv, acc_sh.at[pl.ds(row0 + i * 16, 16)])

    pltpu.sync_copy(zv.at[pl.ds(0, 8)], acc_sh.at[pl.ds(row0 + 624, 8)])
    plsc.subcore_barrier()

    def _fire_idx(p, ib, sem):
        return pltpu.async_copy(idx_hbm.at[wid, p], ib, sem)

    def _wait_idx(p, ib, sem):
        pltpu.make_async_copy(idx_hbm.at[wid, p], ib, sem).wait()

    def _fire_gather(ib, r, rows, sem):
        return pltpu.async_copy(h_hbm.at[ib.at[r]], rows, sem)

    def _drain(ib, r, rows, sem):
        pltpu.make_async_copy(h_hbm.at[ib.at[r - 1]], rows, sem).wait()
        pltpu.sync_copy(rows, acc_sh.at[ib.at[r]], add=True)

    npair = NCH // 2
    _fire_idx(0, ib_a, sem_ia)
    _wait_idx(0, ib_a, sem_ia)
    _fire_gather(ib_a, 0, rows_a, sem_a)
    _fire_idx(1, ib_b, sem_ib)

    def _pair(p, A, B, sa, sb, sia, sib):
        _fire_gather(A, 2, rows_b, sb)
        _drain(A, 1, rows_a, sa)
        _wait_idx(p + 1, B, sib)
        _fire_gather(B, 0, rows_a, sa)
        _drain(A, 3, rows_b, sb)
        _fire_idx(p + 2, A, sia)

    @pl.loop(0, npair // 2 - 1)
    def _edge(k):
        p = 2 * k
        _pair(p, ib_a, ib_b, sem_a, sem_b, sem_ia, sem_ib)
        _pair(p + 1, ib_b, ib_a, sem_a, sem_b, sem_ib, sem_ia)

    _fire_gather(ib_a, 2, rows_b, sem_b)
    _drain(ib_a, 1, rows_a, sem_a)
    _wait_idx(npair - 1, ib_b, sem_ib)
    _fire_gather(ib_b, 0, rows_a, sem_a)
    _drain(ib_a, 3, rows_b, sem_b)
    _fire_gather(ib_b, 2, rows_b, sem_b)
    _drain(ib_b, 1, rows_a, sem_a)
    _drain(ib_b, 3, rows_b, sem_b)

    plsc.subcore_barrier()
    pltpu.sync_copy(acc_sh.at[pl.ds(row0, RPS)],
                    out_hbm.at[c, pl.ds(row0, RPS)])


_sc_agg = functools.partial(
    pl.kernel,
    out_type=jax.ShapeDtypeStruct((NC, NP, FEAT), jnp.float32),
    mesh=plsc.VectorSubcoreMesh(core_axis_name="c", subcore_axis_name="s"),
    scratch_types=[
        pltpu.VMEM((4, CHUNK), jnp.int32),
        pltpu.VMEM((4, CHUNK), jnp.int32),
        pltpu.VMEM((CHUNK, FEAT), jnp.float32),
        pltpu.VMEM((CHUNK, FEAT), jnp.float32),
        pltpu.VMEM((16, FEAT), jnp.float32),
        pltpu.VMEM_SHARED((NP, FEAT), jnp.float32),
        pltpu.SemaphoreType.DMA,
        pltpu.SemaphoreType.DMA,
        pltpu.SemaphoreType.DMA,
        pltpu.SemaphoreType.DMA,
    ],
)(_sc_agg_body)


def _tc_input_body(x_ref, w_ref, b_ref, o_ref):
    h = jnp.dot(x_ref[...], w_ref[...], preferred_element_type=jnp.float32)
    o_ref[0:N, :] = jnp.maximum(h + b_ref[...], 0.0)


def _tc_input(x, w0, b0):
    return pl.pallas_call(
        _tc_input_body,
        out_shape=jax.ShapeDtypeStruct((NP, FEAT), jnp.float32),
    )(x, w0, b0)


def _bn_relu(z, g, b):
    m = jnp.sum(z, axis=0, keepdims=True) * (1.0 / N)
    d = z - m
    v = jnp.sum(d * d, axis=0, keepdims=True) * (1.0 / N)
    return jnp.maximum(g * d * jax.lax.rsqrt(v + 1e-5) + b, 0.0)


def _tc_dense_body(h_ref, p_ref, eps_ref, w1_ref, b1_ref, g1_ref, be1_ref,
                   w2_ref, b2_ref, g2_ref, be2_ref, o_ref):
    h = h_ref[0:N, :]
    agg = p_ref[0, 0:N, :] + p_ref[1, 0:N, :]
    z = (1.0 + eps_ref[0, 0]) * h + agg
    z = jnp.dot(z, w1_ref[...], preferred_element_type=jnp.float32) + b1_ref[...]
    z = _bn_relu(z, g1_ref[...], be1_ref[...])
    z = jnp.dot(z, w2_ref[...], preferred_element_type=jnp.float32) + b2_ref[...]
    z = _bn_relu(z, g2_ref[...], be2_ref[...])
    o_ref[0:N, :] = z + h


def _tc_dense(h, parts, eps_l, w1, b1, g1, be1, w2, b2, g2, be2):
    return pl.pallas_call(
        _tc_dense_body,
        out_shape=jax.ShapeDtypeStruct((NP, FEAT), jnp.float32),
    )(h, parts, eps_l, w1, b1, g1, be1, w2, b2, g2, be2)


def _tc_pool_body(h_ref, batch_ref, wout_ref, bout_ref, o_ref):
    h = h_ref[0:N, :]
    ids = batch_ref[...]
    onehot = (ids == lax.broadcasted_iota(jnp.int32, (N, G), 1))
    onehot = onehot.astype(jnp.float32)
    cdims = (((0,), (0,)), ((), ()))
    sums = lax.dot_general(onehot, h, cdims,
                           preferred_element_type=jnp.float32)
    cnt = lax.dot_general(onehot, jnp.ones((N, 1), jnp.float32), cdims,
                          preferred_element_type=jnp.float32)
    pooled = sums / jnp.maximum(cnt, 1.0)
    o_ref[...] = jnp.dot(pooled, wout_ref[...],
                         preferred_element_type=jnp.float32) + bout_ref[...]


def _tc_pool(h, batch2d, wout, bout):
    return pl.pallas_call(
        _tc_pool_body,
        out_shape=jax.ShapeDtypeStruct((G, FEAT), jnp.float32),
    )(h, batch2d, wout, bout)


def kernel(x, edge_index, edge_attr, batch, W0, b0, eps, W1s, b1s, g1s, be1s,
           W2s, b2s, gs, bes, Wout, bout):
    padw = EP // NW - E // NW
    pad_rows = (N + (jnp.arange(padw, dtype=jnp.int32) % (NP - N)))
    pad_blk = jnp.broadcast_to(pad_rows, (NW, padw))
    src = jnp.concatenate([edge_index[0].reshape(NW, E // NW), pad_blk], 1)
    dst = jnp.concatenate([edge_index[1].reshape(NW, E // NW), pad_blk], 1)
    src = src.reshape(NW, NCH, CHUNK)
    dst = dst.reshape(NW, NCH, CHUNK)
    idx = jnp.stack([src[:, 0::2], dst[:, 0::2], src[:, 1::2], dst[:, 1::2]],
                    axis=2)

    h = _tc_input(x, W0, b0.reshape(1, FEAT))
    for l in range(2):
        parts = _sc_agg(h, idx)
        h = _tc_dense(h, parts, eps[l].reshape(1, 1),
                      W1s[l], b1s[l].reshape(1, FEAT),
                      g1s[l].reshape(1, FEAT), be1s[l].reshape(1, FEAT),
                      W2s[l], b2s[l].reshape(1, FEAT),
                      gs[l].reshape(1, FEAT), bes[l].reshape(1, FEAT))
    return _tc_pool(h, batch.reshape(N, 1).astype(jnp.int32),
                    Wout, bout.reshape(1, FEAT))


# speedup vs baseline: 3.7765x; 1.0211x over previous
import functools

import jax
import jax.numpy as jnp
from jax import lax
from jax.experimental import pallas as pl
from jax.experimental.pallas import tpu as pltpu
from jax.experimental.pallas import tpu_sc as plsc

N = 10000
NP = 10112
E = 320000
FEAT = 128
G = 128
NC = 2
NS = 16
NW = NC * NS
CHUNK = 128
NCH = 80
EP = NW * NCH * CHUNK
RPS = NP // NS


def _sc_agg_body(h_hbm, idx_hbm, out_hbm,
                 ib_a, ib_b, rows_a, rows_b, zv, acc_sh,
                 sem_a, sem_b, sem_ia, sem_ib):
    c = lax.axis_index("c")
    s = lax.axis_index("s")
    wid = s * NC + c

    zero16 = jnp.zeros((16,), jnp.float32)
    for r in range(16):
        for q in range(8):
            zv[r, pl.ds(q * 16, 16)] = zero16
    row0 = s * RPS

    @pl.loop(0, 39)
    def _zero(i):
        pltpu.sync_copy(zv, acc_sh.at[pl.ds(row0 + i * 16, 16)])

    pltpu.sync_copy(zv.at[pl.ds(0, 8)], acc_sh.at[pl.ds(row0 + 624, 8)])
    plsc.subcore_barrier()

    def _fire_idx(p, ib, sem):
        return pltpu.async_copy(idx_hbm.at[wid, p], ib, sem)

    def _wait_idx(p, ib, sem):
        pltpu.make_async_copy(idx_hbm.at[wid, p], ib, sem).wait()

    def _fire_gather(ib, r, rows, sem):
        return pltpu.async_copy(h_hbm.at[ib.at[r]], rows, sem)

    def _drain(ib, r, rows, sem):
        pltpu.make_async_copy(h_hbm.at[ib.at[r - 1]], rows, sem).wait()
        pltpu.sync_copy(rows, acc_sh.at[ib.at[r]], add=True)

    npair = NCH // 2
    _fire_idx(0, ib_a, sem_ia)
    _wait_idx(0, ib_a, sem_ia)
    _fire_gather(ib_a, 0, rows_a, sem_a)
    _fire_idx(1, ib_b, sem_ib)

    def _pair(p, A, B, sa, sb, sia, sib):
        _fire_gather(A, 2, rows_b, sb)
        _drain(A, 1, rows_a, sa)
        _wait_idx(p + 1, B, sib)
        _fire_gather(B, 0, rows_a, sa)
        _drain(A, 3, rows_b, sb)
        _fire_idx(p + 2, A, sia)

    @pl.loop(0, npair // 2 - 1)
    def _edge(k):
        p = 2 * k
        _pair(p, ib_a, ib_b, sem_a, sem_b, sem_ia, sem_ib)
        _pair(p + 1, ib_b, ib_a, sem_a, sem_b, sem_ib, sem_ia)

    _fire_gather(ib_a, 2, rows_b, sem_b)
    _drain(ib_a, 1, rows_a, sem_a)
    _wait_idx(npair - 1, ib_b, sem_ib)
    _fire_gather(ib_b, 0, rows_a, sem_a)
    _drain(ib_a, 3, rows_b, sem_b)
    _fire_gather(ib_b, 2, rows_b, sem_b)
    _drain(ib_b, 1, rows_a, sem_a)
    _drain(ib_b, 3, rows_b, sem_b)

    plsc.subcore_barrier()
    pltpu.sync_copy(acc_sh.at[pl.ds(row0, RPS)],
                    out_hbm.at[c, pl.ds(row0, RPS)])


_sc_agg = functools.partial(
    pl.kernel,
    out_type=jax.ShapeDtypeStruct((NC, NP, FEAT), jnp.float32),
    mesh=plsc.VectorSubcoreMesh(core_axis_name="c", subcore_axis_name="s"),
    scratch_types=[
        pltpu.VMEM((4, CHUNK), jnp.int32),
        pltpu.VMEM((4, CHUNK), jnp.int32),
        pltpu.VMEM((CHUNK, FEAT), jnp.float32),
        pltpu.VMEM((CHUNK, FEAT), jnp.float32),
        pltpu.VMEM((16, FEAT), jnp.float32),
        pltpu.VMEM_SHARED((NP, FEAT), jnp.float32),
        pltpu.SemaphoreType.DMA,
        pltpu.SemaphoreType.DMA,
        pltpu.SemaphoreType.DMA,
        pltpu.SemaphoreType.DMA,
    ],
)(_sc_agg_body)


def _tc_input_body(x_ref, w_ref, b_ref, o_ref):
    h = jnp.dot(x_ref[...], w_ref[...], preferred_element_type=jnp.float32)
    o_ref[0:N, :] = jnp.maximum(h + b_ref[...], 0.0)


def _tc_input(x, w0, b0):
    return pl.pallas_call(
        _tc_input_body,
        out_shape=jax.ShapeDtypeStruct((NP, FEAT), jnp.float32),
    )(x, w0, b0)


def _bn_relu(z, g, b):
    m = jnp.sum(z, axis=0, keepdims=True) * (1.0 / N)
    d = z - m
    v = jnp.sum(d * d, axis=0, keepdims=True) * (1.0 / N)
    return jnp.maximum(g * d * jax.lax.rsqrt(v + 1e-5) + b, 0.0)


def _tc_dense_body(h_ref, p_ref, eps_ref, w1_ref, b1_ref, g1_ref, be1_ref,
                   w2_ref, b2_ref, g2_ref, be2_ref, o_ref):
    h = h_ref[0:N, :]
    agg = p_ref[0, 0:N, :] + p_ref[1, 0:N, :]
    z = (1.0 + eps_ref[0, 0]) * h + agg
    z = jnp.dot(z, w1_ref[...], preferred_element_type=jnp.float32) + b1_ref[...]
    z = _bn_relu(z, g1_ref[...], be1_ref[...])
    z = jnp.dot(z, w2_ref[...], preferred_element_type=jnp.float32) + b2_ref[...]
    z = _bn_relu(z, g2_ref[...], be2_ref[...])
    o_ref[0:N, :] = z + h


def _tc_dense(h, parts, eps_l, w1, b1, g1, be1, w2, b2, g2, be2):
    return pl.pallas_call(
        _tc_dense_body,
        out_shape=jax.ShapeDtypeStruct((NP, FEAT), jnp.float32),
    )(h, parts, eps_l, w1, b1, g1, be1, w2, b2, g2, be2)


def _tc_last_body(h_ref, p_ref, eps_ref, w1_ref, b1_ref, g1_ref, be1_ref,
                  w2_ref, b2_ref, g2_ref, be2_ref,
                  batch_ref, wout_ref, bout_ref, o_ref):
    h = h_ref[0:N, :]
    agg = p_ref[0, 0:N, :] + p_ref[1, 0:N, :]
    z = (1.0 + eps_ref[0, 0]) * h + agg
    z = jnp.dot(z, w1_ref[...], preferred_element_type=jnp.float32) + b1_ref[...]
    z = _bn_relu(z, g1_ref[...], be1_ref[...])
    z = jnp.dot(z, w2_ref[...], preferred_element_type=jnp.float32) + b2_ref[...]
    z = _bn_relu(z, g2_ref[...], be2_ref[...])
    h = z + h
    ids = batch_ref[...]
    onehot = (ids == lax.broadcasted_iota(jnp.int32, (N, G), 1))
    onehot = onehot.astype(jnp.float32)
    cdims = (((0,), (0,)), ((), ()))
    sums = lax.dot_general(onehot, h, cdims,
                           preferred_element_type=jnp.float32)
    cnt = lax.dot_general(onehot, jnp.ones((N, 1), jnp.float32), cdims,
                          preferred_element_type=jnp.float32)
    pooled = sums / jnp.maximum(cnt, 1.0)
    o_ref[...] = jnp.dot(pooled, wout_ref[...],
                         preferred_element_type=jnp.float32) + bout_ref[...]


def _tc_last(h, parts, eps_l, w1, b1, g1, be1, w2, b2, g2, be2,
             batch2d, wout, bout):
    return pl.pallas_call(
        _tc_last_body,
        out_shape=jax.ShapeDtypeStruct((G, FEAT), jnp.float32),
    )(h, parts, eps_l, w1, b1, g1, be1, w2, b2, g2, be2,
      batch2d, wout, bout)


def kernel(x, edge_index, edge_attr, batch, W0, b0, eps, W1s, b1s, g1s, be1s,
           W2s, b2s, gs, bes, Wout, bout):
    padw = EP // NW - E // NW
    pad_rows = (N + (jnp.arange(padw, dtype=jnp.int32) % (NP - N)))
    pad_blk = jnp.broadcast_to(pad_rows, (NW, padw))
    src = jnp.concatenate([edge_index[0].reshape(NW, E // NW), pad_blk], 1)
    dst = jnp.concatenate([edge_index[1].reshape(NW, E // NW), pad_blk], 1)
    src = src.reshape(NW, NCH, CHUNK)
    dst = dst.reshape(NW, NCH, CHUNK)
    idx = jnp.stack([src[:, 0::2], dst[:, 0::2], src[:, 1::2], dst[:, 1::2]],
                    axis=2)

    h = _tc_input(x, W0, b0.reshape(1, FEAT))
    parts = _sc_agg(h, idx)
    h = _tc_dense(h, parts, eps[0].reshape(1, 1),
                  W1s[0], b1s[0].reshape(1, FEAT),
                  g1s[0].reshape(1, FEAT), be1s[0].reshape(1, FEAT),
                  W2s[0], b2s[0].reshape(1, FEAT),
                  gs[0].reshape(1, FEAT), bes[0].reshape(1, FEAT))
    parts = _sc_agg(h, idx)
    return _tc_last(h, parts, eps[1].reshape(1, 1),
                    W1s[1], b1s[1].reshape(1, FEAT),
                    g1s[1].reshape(1, FEAT), be1s[1].reshape(1, FEAT),
                    W2s[1], b2s[1].reshape(1, FEAT),
                    gs[1].reshape(1, FEAT), bes[1].reshape(1, FEAT),
                    batch.reshape(N, 1).astype(jnp.int32),
                    Wout, bout.reshape(1, FEAT))


# speedup vs baseline: 3.8273x; 1.0135x over previous
import functools

import jax
import jax.numpy as jnp
from jax import lax
from jax.experimental import pallas as pl
from jax.experimental.pallas import tpu as pltpu
from jax.experimental.pallas import tpu_sc as plsc

N = 10000
NP = 10112
E = 320000
FEAT = 128
G = 128
NC = 2
NS = 16
NW = NC * NS
CHUNK = 128
NCH = 80
EP = NW * NCH * CHUNK
RPS = NP // NS


def _sc_agg_body(h_hbm, idx_hbm, out_hbm,
                 ib_a, ib_b, rows_a, rows_b, zv, acc_sh,
                 sem_a, sem_b, sem_ia, sem_ib, sem_z):
    c = lax.axis_index("c")
    s = lax.axis_index("s")
    wid = s * NC + c

    zero16 = jnp.zeros((16,), jnp.float32)

    @pl.loop(0, 64)
    def _zfill(r):
        for q in range(8):
            zv[r, pl.ds(q * 16, 16)] = zero16

    row0 = s * RPS
    for i in range(9):
        pltpu.async_copy(zv, acc_sh.at[pl.ds(row0 + i * 64, 64)], sem_z)
    pltpu.async_copy(zv.at[pl.ds(0, 56)],
                     acc_sh.at[pl.ds(row0 + 576, 56)], sem_z)

    def _fire_idx(p, ib, sem):
        return pltpu.async_copy(idx_hbm.at[wid, p], ib, sem)

    def _wait_idx(p, ib, sem):
        pltpu.make_async_copy(idx_hbm.at[wid, p], ib, sem).wait()

    def _fire_gather(ib, r, rows, sem):
        return pltpu.async_copy(h_hbm.at[ib.at[r]], rows, sem)

    def _drain(ib, r, rows, sem):
        pltpu.make_async_copy(h_hbm.at[ib.at[r - 1]], rows, sem).wait()
        pltpu.sync_copy(rows, acc_sh.at[ib.at[r]], add=True)

    npair = NCH // 2
    _fire_idx(0, ib_a, sem_ia)

    for i in range(9):
        pltpu.make_async_copy(zv, acc_sh.at[pl.ds(row0, 64)], sem_z).wait()
    pltpu.make_async_copy(zv.at[pl.ds(0, 56)],
                          acc_sh.at[pl.ds(row0, 56)], sem_z).wait()

    _wait_idx(0, ib_a, sem_ia)
    _fire_gather(ib_a, 0, rows_a, sem_a)
    _fire_idx(1, ib_b, sem_ib)
    plsc.subcore_barrier()

    def _pair(p, A, B, sa, sb, sia, sib):
        _fire_gather(A, 2, rows_b, sb)
        _drain(A, 1, rows_a, sa)
        _wait_idx(p + 1, B, sib)
        _fire_gather(B, 0, rows_a, sa)
        _drain(A, 3, rows_b, sb)
        _fire_idx(p + 2, A, sia)

    @pl.loop(0, npair // 2 - 1)
    def _edge(k):
        p = 2 * k
        _pair(p, ib_a, ib_b, sem_a, sem_b, sem_ia, sem_ib)
        _pair(p + 1, ib_b, ib_a, sem_a, sem_b, sem_ib, sem_ia)

    _fire_gather(ib_a, 2, rows_b, sem_b)
    _drain(ib_a, 1, rows_a, sem_a)
    _wait_idx(npair - 1, ib_b, sem_ib)
    _fire_gather(ib_b, 0, rows_a, sem_a)
    _drain(ib_a, 3, rows_b, sem_b)
    _fire_gather(ib_b, 2, rows_b, sem_b)
    _drain(ib_b, 1, rows_a, sem_a)
    _drain(ib_b, 3, rows_b, sem_b)

    plsc.subcore_barrier()
    pltpu.sync_copy(acc_sh.at[pl.ds(row0, RPS)],
                    out_hbm.at[c, pl.ds(row0, RPS)])


_sc_agg = functools.partial(
    pl.kernel,
    out_type=jax.ShapeDtypeStruct((NC, NP, FEAT), jnp.float32),
    mesh=plsc.VectorSubcoreMesh(core_axis_name="c", subcore_axis_name="s"),
    scratch_types=[
        pltpu.VMEM((4, CHUNK), jnp.int32),
        pltpu.VMEM((4, CHUNK), jnp.int32),
        pltpu.VMEM((CHUNK, FEAT), jnp.float32),
        pltpu.VMEM((CHUNK, FEAT), jnp.float32),
        pltpu.VMEM((64, FEAT), jnp.float32),
        pltpu.VMEM_SHARED((NP, FEAT), jnp.float32),
        pltpu.SemaphoreType.DMA,
        pltpu.SemaphoreType.DMA,
        pltpu.SemaphoreType.DMA,
        pltpu.SemaphoreType.DMA,
        pltpu.SemaphoreType.DMA,
    ],
)(_sc_agg_body)


def _tc_input_body(x_ref, w_ref, b_ref, o_ref):
    h = jnp.dot(x_ref[...], w_ref[...], preferred_element_type=jnp.float32)
    o_ref[0:N, :] = jnp.maximum(h + b_ref[...], 0.0)


def _tc_input(x, w0, b0):
    return pl.pallas_call(
        _tc_input_body,
        out_shape=jax.ShapeDtypeStruct((NP, FEAT), jnp.float32),
    )(x, w0, b0)


def _bn_relu(z, g, b):
    m = jnp.sum(z, axis=0, keepdims=True) * (1.0 / N)
    d = z - m
    v = jnp.sum(d * d, axis=0, keepdims=True) * (1.0 / N)
    return jnp.maximum(g * d * jax.lax.rsqrt(v + 1e-5) + b, 0.0)


def _tc_dense_body(h_ref, p_ref, eps_ref, w1_ref, b1_ref, g1_ref, be1_ref,
                   w2_ref, b2_ref, g2_ref, be2_ref, o_ref):
    h = h_ref[0:N, :]
    agg = p_ref[0, 0:N, :] + p_ref[1, 0:N, :]
    z = (1.0 + eps_ref[0, 0]) * h + agg
    z = jnp.dot(z, w1_ref[...], preferred_element_type=jnp.float32) + b1_ref[...]
    z = _bn_relu(z, g1_ref[...], be1_ref[...])
    z = jnp.dot(z, w2_ref[...], preferred_element_type=jnp.float32) + b2_ref[...]
    z = _bn_relu(z, g2_ref[...], be2_ref[...])
    o_ref[0:N, :] = z + h


def _tc_dense(h, parts, eps_l, w1, b1, g1, be1, w2, b2, g2, be2):
    return pl.pallas_call(
        _tc_dense_body,
        out_shape=jax.ShapeDtypeStruct((NP, FEAT), jnp.float32),
    )(h, parts, eps_l, w1, b1, g1, be1, w2, b2, g2, be2)


def _tc_last_body(h_ref, p_ref, eps_ref, w1_ref, b1_ref, g1_ref, be1_ref,
                  w2_ref, b2_ref, g2_ref, be2_ref,
                  batch_ref, wout_ref, bout_ref, o_ref):
    h = h_ref[0:N, :]
    agg = p_ref[0, 0:N, :] + p_ref[1, 0:N, :]
    z = (1.0 + eps_ref[0, 0]) * h + agg
    z = jnp.dot(z, w1_ref[...], preferred_element_type=jnp.float32) + b1_ref[...]
    z = _bn_relu(z, g1_ref[...], be1_ref[...])
    z = jnp.dot(z, w2_ref[...], preferred_element_type=jnp.float32) + b2_ref[...]
    z = _bn_relu(z, g2_ref[...], be2_ref[...])
    h = z + h
    ids = batch_ref[...]
    onehot = (ids == lax.broadcasted_iota(jnp.int32, (N, G), 1))
    onehot = onehot.astype(jnp.float32)
    cdims = (((0,), (0,)), ((), ()))
    sums = lax.dot_general(onehot, h, cdims,
                           preferred_element_type=jnp.float32)
    cnt = lax.dot_general(onehot, jnp.ones((N, 1), jnp.float32), cdims,
                          preferred_element_type=jnp.float32)
    pooled = sums / jnp.maximum(cnt, 1.0)
    o_ref[...] = jnp.dot(pooled, wout_ref[...],
                         preferred_element_type=jnp.float32) + bout_ref[...]


def _tc_last(h, parts, eps_l, w1, b1, g1, be1, w2, b2, g2, be2,
             batch2d, wout, bout):
    return pl.pallas_call(
        _tc_last_body,
        out_shape=jax.ShapeDtypeStruct((G, FEAT), jnp.float32),
    )(h, parts, eps_l, w1, b1, g1, be1, w2, b2, g2, be2,
      batch2d, wout, bout)


def kernel(x, edge_index, edge_attr, batch, W0, b0, eps, W1s, b1s, g1s, be1s,
           W2s, b2s, gs, bes, Wout, bout):
    padw = EP // NW - E // NW
    pad_rows = (N + (jnp.arange(padw, dtype=jnp.int32) % (NP - N)))
    pad_blk = jnp.broadcast_to(pad_rows, (NW, padw))
    src = jnp.concatenate([edge_index[0].reshape(NW, E // NW), pad_blk], 1)
    dst = jnp.concatenate([edge_index[1].reshape(NW, E // NW), pad_blk], 1)
    src = src.reshape(NW, NCH, CHUNK)
    dst = dst.reshape(NW, NCH, CHUNK)
    idx = jnp.stack([src[:, 0::2], dst[:, 0::2], src[:, 1::2], dst[:, 1::2]],
                    axis=2)

    h = _tc_input(x, W0, b0.reshape(1, FEAT))
    parts = _sc_agg(h, idx)
    h = _tc_dense(h, parts, eps[0].reshape(1, 1),
                  W1s[0], b1s[0].reshape(1, FEAT),
                  g1s[0].reshape(1, FEAT), be1s[0].reshape(1, FEAT),
                  W2s[0], b2s[0].reshape(1, FEAT),
                  gs[0].reshape(1, FEAT), bes[0].reshape(1, FEAT))
    parts = _sc_agg(h, idx)
    return _tc_last(h, parts, eps[1].reshape(1, 1),
                    W1s[1], b1s[1].reshape(1, FEAT),
                    g1s[1].reshape(1, FEAT), be1s[1].reshape(1, FEAT),
                    W2s[1], b2s[1].reshape(1, FEAT),
                    gs[1].reshape(1, FEAT), bes[1].reshape(1, FEAT),
                    batch.reshape(N, 1).astype(jnp.int32),
                    Wout, bout.reshape(1, FEAT))


# speedup vs baseline: 3.8291x; 1.0005x over previous
import functools

import jax
import jax.numpy as jnp
from jax import lax
from jax.experimental import pallas as pl
from jax.experimental.pallas import tpu as pltpu
from jax.experimental.pallas import tpu_sc as plsc

N = 10000
NP = 10112
E = 320000
FEAT = 128
G = 128
NC = 2
NS = 16
NW = NC * NS
CHUNK = 128
NCH = 80
EP = NW * NCH * CHUNK
RPS = NP // NS


def _sc_agg_body(h_hbm, idx_hbm, out_hbm,
                 ib_a, ib_b, rows_a, rows_b, zv, acc_sh,
                 sem_a, sem_b, sem_ia, sem_ib, sem_z):
    c = lax.axis_index("c")
    s = lax.axis_index("s")
    wid = s * NC + c

    zero16 = jnp.zeros((16,), jnp.float32)

    @pl.loop(0, 64)
    def _zfill(r):
        for q in range(8):
            zv[r, pl.ds(q * 16, 16)] = zero16

    row0 = s * RPS
    for i in range(9):
        pltpu.async_copy(zv, acc_sh.at[pl.ds(row0 + i * 64, 64)], sem_z)
    pltpu.async_copy(zv.at[pl.ds(0, 56)],
                     acc_sh.at[pl.ds(row0 + 576, 56)], sem_z)

    def _fire_idx(p, ib, sem):
        return pltpu.async_copy(idx_hbm.at[wid, p], ib, sem)

    def _wait_idx(p, ib, sem):
        pltpu.make_async_copy(idx_hbm.at[wid, p], ib, sem).wait()

    def _fire_gather(ib, r, rows, sem):
        return pltpu.async_copy(h_hbm.at[ib.at[r]], rows, sem)

    def _drain(ib, r, rows, sem):
        pltpu.make_async_copy(h_hbm.at[ib.at[r - 1]], rows, sem).wait()
        pltpu.sync_copy(rows, acc_sh.at[ib.at[r]], add=True)

    npair = NCH // 2
    _fire_idx(0, ib_a, sem_ia)

    for i in range(9):
        pltpu.make_async_copy(zv, acc_sh.at[pl.ds(row0, 64)], sem_z).wait()
    pltpu.make_async_copy(zv.at[pl.ds(0, 56)],
                          acc_sh.at[pl.ds(row0, 56)], sem_z).wait()

    _wait_idx(0, ib_a, sem_ia)
    _fire_gather(ib_a, 0, rows_a, sem_a)
    _fire_idx(1, ib_b, sem_ib)
    plsc.subcore_barrier()

    def _pair(p, A, B, sa, sb, sia, sib):
        _fire_gather(A, 2, rows_b, sb)
        _drain(A, 1, rows_a, sa)
        _wait_idx(p + 1, B, sib)
        _fire_gather(B, 0, rows_a, sa)
        _drain(A, 3, rows_b, sb)
        _fire_idx(p + 2, A, sia)

    @pl.loop(0, npair // 2 - 1)
    def _edge(k):
        p = 2 * k
        _pair(p, ib_a, ib_b, sem_a, sem_b, sem_ia, sem_ib)
        _pair(p + 1, ib_b, ib_a, sem_a, sem_b, sem_ib, sem_ia)

    _fire_gather(ib_a, 2, rows_b, sem_b)
    _drain(ib_a, 1, rows_a, sem_a)
    _wait_idx(npair - 1, ib_b, sem_ib)
    _fire_gather(ib_b, 0, rows_a, sem_a)
    _drain(ib_a, 3, rows_b, sem_b)
    _fire_gather(ib_b, 2, rows_b, sem_b)
    _drain(ib_b, 1, rows_a, sem_a)
    _drain(ib_b, 3, rows_b, sem_b)

    plsc.subcore_barrier()
    pltpu.sync_copy(acc_sh.at[pl.ds(row0, RPS)],
                    out_hbm.at[c, pl.ds(row0, RPS)])


_sc_agg = functools.partial(
    pl.kernel,
    out_type=jax.ShapeDtypeStruct((NC, NP, FEAT), jnp.float32),
    mesh=plsc.VectorSubcoreMesh(core_axis_name="c", subcore_axis_name="s"),
    scratch_types=[
        pltpu.VMEM((4, CHUNK), jnp.int32),
        pltpu.VMEM((4, CHUNK), jnp.int32),
        pltpu.VMEM((CHUNK, FEAT), jnp.float32),
        pltpu.VMEM((CHUNK, FEAT), jnp.float32),
        pltpu.VMEM((64, FEAT), jnp.float32),
        pltpu.VMEM_SHARED((NP, FEAT), jnp.float32),
        pltpu.SemaphoreType.DMA,
        pltpu.SemaphoreType.DMA,
        pltpu.SemaphoreType.DMA,
        pltpu.SemaphoreType.DMA,
        pltpu.SemaphoreType.DMA,
    ],
)(_sc_agg_body)


def _tc_input_body(x_ref, w_ref, b_ref, o_ref):
    h = jnp.dot(x_ref[...], w_ref[...], preferred_element_type=jnp.float32)
    o_ref[0:N, :] = jnp.maximum(h + b_ref[...], 0.0)


def _tc_input(x, w0, b0):
    return pl.pallas_call(
        _tc_input_body,
        out_shape=jax.ShapeDtypeStruct((NP, FEAT), jnp.float32),
    )(x, w0, b0)


def _bn_relu(z, g, b):
    m = jnp.sum(z, axis=0, keepdims=True) * (1.0 / N)
    d = z - m
    v = jnp.sum(d * d, axis=0, keepdims=True) * (1.0 / N)
    return jnp.maximum(g * d * jax.lax.rsqrt(v + 1e-5) + b, 0.0)


def _tc_dense_body(h_ref, p_ref, eps_ref, w1_ref, b1_ref, g1_ref, be1_ref,
                   w2_ref, b2_ref, g2_ref, be2_ref, o_ref):
    h = h_ref[0:N, :]
    agg = p_ref[0, 0:N, :] + p_ref[1, 0:N, :]
    z = (1.0 + eps_ref[0, 0]) * h + agg
    z = jnp.dot(z, w1_ref[...], preferred_element_type=jnp.float32) + b1_ref[...]
    z = _bn_relu(z, g1_ref[...], be1_ref[...])
    z = jnp.dot(z, w2_ref[...], preferred_element_type=jnp.float32) + b2_ref[...]
    z = _bn_relu(z, g2_ref[...], be2_ref[...])
    o_ref[0:N, :] = z + h


def _tc_dense(h, parts, eps_l, w1, b1, g1, be1, w2, b2, g2, be2):
    return pl.pallas_call(
        _tc_dense_body,
        out_shape=jax.ShapeDtypeStruct((NP, FEAT), jnp.float32),
    )(h, parts, eps_l, w1, b1, g1, be1, w2, b2, g2, be2)


def _tc_last_body(h_ref, p_ref, eps_ref, w1_ref, b1_ref, g1_ref, be1_ref,
                  w2_ref, b2_ref, g2_ref, be2_ref,
                  batch_ref, wout_ref, bout_ref, o_ref):
    h = h_ref[0:N, :]
    agg = p_ref[0, 0:N, :] + p_ref[1, 0:N, :]
    z = (1.0 + eps_ref[0, 0]) * h + agg
    z = jnp.dot(z, w1_ref[...], preferred_element_type=jnp.float32) + b1_ref[...]
    z = _bn_relu(z, g1_ref[...], be1_ref[...])
    z = jnp.dot(z, w2_ref[...], preferred_element_type=jnp.float32) + b2_ref[...]
    z = _bn_relu(z, g2_ref[...], be2_ref[...])
    h = z + h
    ids = batch_ref[...]
    onehot = (ids == lax.broadcasted_iota(jnp.int32, (N, G), 1))
    onehot = onehot.astype(jnp.float32)
    cdims = (((0,), (0,)), ((), ()))
    sums = lax.dot_general(onehot, h, cdims,
                           preferred_element_type=jnp.float32)
    cnt = lax.dot_general(onehot, jnp.ones((N, 1), jnp.float32), cdims,
                          preferred_element_type=jnp.float32)
    pooled = sums / jnp.maximum(cnt, 1.0)
    o_ref[...] = jnp.dot(pooled, wout_ref[...],
                         preferred_element_type=jnp.float32) + bout_ref[...]


def _tc_last(h, parts, eps_l, w1, b1, g1, be1, w2, b2, g2, be2,
             batch2d, wout, bout):
    return pl.pallas_call(
        _tc_last_body,
        out_shape=jax.ShapeDtypeStruct((G, FEAT), jnp.float32),
    )(h, parts, eps_l, w1, b1, g1, be1, w2, b2, g2, be2,
      batch2d, wout, bout)


def kernel(x, edge_index, edge_attr, batch, W0, b0, eps, W1s, b1s, g1s, be1s,
           W2s, b2s, gs, bes, Wout, bout):
    ei = edge_index.astype(jnp.int32)
    padw = EP // NW - E // NW
    pad_rows = (N + (jnp.arange(padw, dtype=jnp.int32) % (NP - N)))
    pad_blk = jnp.broadcast_to(pad_rows, (NW, padw))
    src = jnp.concatenate([ei[0].reshape(NW, E // NW), pad_blk], 1)
    dst = jnp.concatenate([ei[1].reshape(NW, E // NW), pad_blk], 1)
    src = src.reshape(NW, NCH, CHUNK)
    dst = dst.reshape(NW, NCH, CHUNK)
    idx = jnp.stack([src[:, 0::2], dst[:, 0::2], src[:, 1::2], dst[:, 1::2]],
                    axis=2)

    h = _tc_input(x, W0, b0.reshape(1, FEAT))
    parts = _sc_agg(h, idx)
    h = _tc_dense(h, parts, eps[0].reshape(1, 1),
                  W1s[0], b1s[0].reshape(1, FEAT),
                  g1s[0].reshape(1, FEAT), be1s[0].reshape(1, FEAT),
                  W2s[0], b2s[0].reshape(1, FEAT),
                  gs[0].reshape(1, FEAT), bes[0].reshape(1, FEAT))
    parts = _sc_agg(h, idx)
    return _tc_last(h, parts, eps[1].reshape(1, 1),
                    W1s[1], b1s[1].reshape(1, FEAT),
                    g1s[1].reshape(1, FEAT), be1s[1].reshape(1, FEAT),
                    W2s[1], b2s[1].reshape(1, FEAT),
                    gs[1].reshape(1, FEAT), bes[1].reshape(1, FEAT),
                    batch.reshape(N, 1).astype(jnp.int32),
                    Wout, bout.reshape(1, FEAT))
